# baseline (speedup 1.0000x reference)
import functools

import jax
import jax.numpy as jnp
from jax import lax
from jax.experimental import pallas as pl
from jax.experimental.pallas import tpu as pltpu
from jax.experimental.pallas import tpu_sc as plsc

F32 = jnp.float32
G = 128
ROWS = 2000


def _segsum_call(h, src, dst):
    N, D = h.shape
    E = src.shape[0]
    NC, NS = 2, 16
    NW = NC * NS
    per_tile = E // NW
    CHUNK = 40
    NCH = per_tile // CHUNK
    NB = 5
    NPAD = -(-N // (NS * 8)) * NS * 8
    rpt = NPAD // NS
    mesh = plsc.VectorSubcoreMesh(core_axis_name="c", subcore_axis_name="s")

    sd = jnp.stack([src.reshape(NW * NCH, CHUNK),
                    dst.reshape(NW * NCH, CHUNK)], axis=1)
    zeros = jnp.zeros((rpt, D), F32)

    @functools.partial(
        pl.kernel,
        mesh=mesh,
        out_type=jax.ShapeDtypeStruct((NC, NPAD, D), F32),
        scratch_types=[
            pltpu.VMEM((NB, 2, CHUNK), jnp.int32),
            pltpu.VMEM((NB, CHUNK, D), F32),
            pltpu.VMEM_SHARED((NPAD, D), F32),
            pltpu.SemaphoreType.DMA,
            pltpu.SemaphoreType.DMA,
            pltpu.SemaphoreType.DMA,
        ],
    )
    def segsum(h_hbm, sd_hbm, z_hbm, out_hbm, sd_v, rows_v, acc_sh,
               isem, gsem, ssem):
        c = lax.axis_index("c")
        s = lax.axis_index("s")
        w = c * NS + s

        r0 = s * rpt
        pltpu.sync_copy(z_hbm, acc_sh.at[pl.ds(r0, rpt)])
        plsc.subcore_barrier()

        def body(g, carry):
            k0 = w * NCH + g * NB
            icp = [pltpu.async_copy(sd_hbm.at[k0 + b], sd_v.at[b], isem)
                   for b in range(NB)]
            for cp in icp:
                cp.wait()
            gcp = [pltpu.async_copy(h_hbm.at[sd_v.at[b, 0]], rows_v.at[b],
                                    gsem)
                   for b in range(NB)]
            for cp in gcp:
                cp.wait()
            scp = [pltpu.async_copy(rows_v.at[b], acc_sh.at[sd_v.at[b, 1]],
                                    ssem, add=True)
                   for b in range(NB)]
            for cp in scp:
                cp.wait()
            return carry

        lax.fori_loop(0, NCH // NB, body, 0)
        plsc.subcore_barrier()
        pltpu.sync_copy(acc_sh.at[pl.ds(r0, rpt)],
                        out_hbm.at[c, pl.ds(r0, rpt)])

    return segsum(h, sd, zeros)


def _enc_body(x_ref, w_ref, b_ref, v_ref, o_ref):
    o_ref[...] = (jnp.dot(x_ref[...], w_ref[...], preferred_element_type=F32)
                  + b_ref[...] + v_ref[...])


def _enc_call(x, W, b, v):
    N, DI = x.shape
    EMB = W.shape[1]
    return pl.pallas_call(
        _enc_body,
        grid=(N // ROWS,),
        in_specs=[
            pl.BlockSpec((ROWS, DI), lambda i: (i, 0)),
            pl.BlockSpec((DI, EMB), lambda i: (0, 0)),
            pl.BlockSpec((1, EMB), lambda i: (0, 0)),
            pl.BlockSpec((1, EMB), lambda i: (0, 0)),
        ],
        out_specs=pl.BlockSpec((ROWS, EMB), lambda i: (i, 0)),
        out_shape=jax.ShapeDtypeStruct((N, EMB), F32),
    )(x, W, b, v)


def _onehot(bat_block):
    return (bat_block == lax.broadcasted_iota(jnp.int32, (ROWS, G), 1)
            ).astype(F32)


def _layer_body(hp, agg, bat, vn, w1, b1, w2, b2, ep, vw1, vb1, vw2, vb2,
                vnn, hn, zbuf, pooled, vns):
    p = pl.program_id(0)
    i = pl.program_id(1)
    oh = _onehot(bat[...])

    @pl.when(p == 0)
    def _():
        a = agg[0] + agg[1]
        u = hp[...] * ep[...] + a
        t = jnp.maximum(jnp.dot(u, w1[...], preferred_element_type=F32)
                        + b1[...], 0.0)
        z = jnp.dot(t, w2[...], preferred_element_type=F32) + b2[...]
        z = jnp.maximum(z, 0.0)
        zbuf[pl.ds(i * ROWS, ROWS), :] = z
        pp = lax.dot_general(oh, z, (((0,), (0,)), ((), ())),
                             preferred_element_type=F32)

        @pl.when(i == 0)
        def _():
            pooled[...] = pp

        @pl.when(i > 0)
        def _():
            pooled[...] += pp

    @pl.when((p == 1) & (i == 0))
    def _():
        vt = pooled[...] + vn[...]
        t = jnp.maximum(jnp.dot(vt, vw1[...], preferred_element_type=F32)
                        + vb1[...], 0.0)
        v2 = jnp.maximum(jnp.dot(t, vw2[...], preferred_element_type=F32)
                         + vb2[...], 0.0)
        vns[...] = v2
        vnn[...] = v2

    @pl.when(p == 1)
    def _():
        z = zbuf[pl.ds(i * ROWS, ROWS), :]
        hn[...] = z + jnp.dot(oh, vns[...], preferred_element_type=F32)


def _layer_call(hp, agg, bat, vn, w1, b1, w2, b2, ep, vw1, vb1, vw2, vb2):
    N, EMB = hp.shape
    HID = w1.shape[1]
    return pl.pallas_call(
        _layer_body,
        grid=(2, N // ROWS),
        in_specs=[
            pl.BlockSpec((ROWS, EMB),
                         lambda p, i: (jnp.where(p == 0, i, 0), 0)),
            pl.BlockSpec((2, ROWS, EMB),
                         lambda p, i: (0, jnp.where(p == 0, i, 0), 0)),
            pl.BlockSpec((ROWS, 1), lambda p, i: (i, 0)),
            pl.BlockSpec((G, EMB), lambda p, i: (0, 0)),
            pl.BlockSpec((EMB, HID), lambda p, i: (0, 0)),
            pl.BlockSpec((1, HID), lambda p, i: (0, 0)),
            pl.BlockSpec((HID, EMB), lambda p, i: (0, 0)),
            pl.BlockSpec((1, EMB), lambda p, i: (0, 0)),
            pl.BlockSpec((1, 1), lambda p, i: (0, 0)),
            pl.BlockSpec((EMB, HID), lambda p, i: (0, 0)),
            pl.BlockSpec((1, HID), lambda p, i: (0, 0)),
            pl.BlockSpec((HID, EMB), lambda p, i: (0, 0)),
            pl.BlockSpec((1, EMB), lambda p, i: (0, 0)),
        ],
        out_specs=[
            pl.BlockSpec((G, EMB), lambda p, i: (0, 0)),
            pl.BlockSpec((ROWS, EMB),
                         lambda p, i: (jnp.where(p == 1, i, 0), 0)),
        ],
        out_shape=[
            jax.ShapeDtypeStruct((G, EMB), F32),
            jax.ShapeDtypeStruct((N, EMB), F32),
        ],
        scratch_shapes=[pltpu.VMEM((N, EMB), F32), pltpu.VMEM((G, EMB), F32),
                        pltpu.VMEM((G, EMB), F32)],
    )(hp, agg, bat, vn, w1, b1, w2, b2, ep, vw1, vb1, vw2, vb2)


def _fin_body(hp, agg, bat, w1, b1, w2, b2, ep, cw, cb, pred, pooled_s, cnt_s):
    i = pl.program_id(0)
    a = agg[0] + agg[1]
    u = hp[...] * ep[...] + a
    t = jnp.maximum(jnp.dot(u, w1[...], preferred_element_type=F32)
                    + b1[...], 0.0)
    z = jnp.dot(t, w2[...], preferred_element_type=F32) + b2[...]
    oh = _onehot(bat[...])
    p = lax.dot_general(oh, z, (((0,), (0,)), ((), ())),
                        preferred_element_type=F32)
    cnt = lax.dot_general(oh, jnp.ones((ROWS, 8), F32),
                          (((0,), (0,)), ((), ())),
                          preferred_element_type=F32)

    @pl.when(i == 0)
    def _():
        pooled_s[...] = p
        cnt_s[...] = cnt

    @pl.when(i > 0)
    def _():
        pooled_s[...] += p
        cnt_s[...] += cnt

    @pl.when(i == pl.num_programs(0) - 1)
    def _():
        rep = pooled_s[...] / jnp.maximum(cnt_s[...][:, :1], 1.0)
        pred[...] = jnp.dot(rep, cw[...], preferred_element_type=F32) + cb[...]


def _fin_call(hp, agg, bat, w1, b1, w2, b2, ep, cw, cb):
    N, EMB = hp.shape
    HID = w1.shape[1]
    OUT = cw.shape[1]
    return pl.pallas_call(
        _fin_body,
        grid=(N // ROWS,),
        in_specs=[
            pl.BlockSpec((ROWS, EMB), lambda i: (i, 0)),
            pl.BlockSpec((2, ROWS, EMB), lambda i: (0, i, 0)),
            pl.BlockSpec((ROWS, 1), lambda i: (i, 0)),
            pl.BlockSpec((EMB, HID), lambda i: (0, 0)),
            pl.BlockSpec((1, HID), lambda i: (0, 0)),
            pl.BlockSpec((HID, EMB), lambda i: (0, 0)),
            pl.BlockSpec((1, EMB), lambda i: (0, 0)),
            pl.BlockSpec((1, 1), lambda i: (0, 0)),
            pl.BlockSpec((EMB, OUT), lambda i: (0, 0)),
            pl.BlockSpec((1, OUT), lambda i: (0, 0)),
        ],
        out_specs=pl.BlockSpec((G, OUT), lambda i: (0, 0)),
        out_shape=jax.ShapeDtypeStruct((G, OUT), F32),
        scratch_shapes=[pltpu.VMEM((G, EMB), F32), pltpu.VMEM((G, 8), F32)],
    )(hp, agg, bat, w1, b1, w2, b2, ep, cw, cb)


def kernel(x, edge_index, batch, enc_W, enc_b, W1, b1, W2, b2, eps, vn0,
           VW1, Vb1, VW2, Vb2, CW, Cb):
    N = x.shape[0]
    EMB = enc_W.shape[1]
    L = W1.shape[0]
    src = edge_index[0]
    dst = edge_index[1]
    bat = batch.reshape(N, 1)
    epsp = (1.0 + eps).reshape(L, 1, 1).astype(F32)

    h = _enc_call(x, enc_W, enc_b.reshape(1, EMB), vn0.reshape(1, EMB))
    vn = jnp.tile(vn0[None, :], (G, 1))
    for l in range(L - 1):
        agg = _segsum_call(h, src, dst)
        vn, h = _layer_call(h, agg, bat, vn, W1[l], b1[l].reshape(1, -1),
                            W2[l], b2[l].reshape(1, -1), epsp[l],
                            VW1[l], Vb1[l].reshape(1, -1),
                            VW2[l], Vb2[l].reshape(1, -1))
    agg = _segsum_call(h, src, dst)
    return _fin_call(h, agg, bat, W1[L - 1], b1[L - 1].reshape(1, -1),
                     W2[L - 1], b2[L - 1].reshape(1, -1), epsp[L - 1],
                     CW, Cb.reshape(1, -1))
